# baseline (speedup 1.0000x reference)
import jax
import jax.numpy as jnp
from jax.experimental import pallas as pl
from jax.experimental.pallas import tpu as pltpu

_G = 4


def _body(x2_ref, A_ref, W_in_ref, b_in_ref, W_ref, b_ref, W_out_ref,
          b_out_ref, out_ref, hw_ref, An_ref):
    T = x2_ref.shape[0]
    H = W_in_ref.shape[1]
    L, C, _ = A_ref.shape
    F = x2_ref.shape[1] // C
    G = _G
    Tg = T // G
    GC = G * C

    @pl.when(pl.program_id(0) == 0)
    def _build_adjacency():
        ri = jax.lax.broadcasted_iota(jnp.int32, (GC, GC), 0)
        ci = jax.lax.broadcasted_iota(jnp.int32, (GC, GC), 1)
        mask = (ri % G) == (ci % G)
        for l in range(L):
            a = A_ref[l].astype(jnp.float32)
            a = a - jnp.max(a, axis=-1, keepdims=True)
            e = jnp.exp(a)
            An = e / jnp.sum(e, axis=-1, keepdims=True)
            An_rep = jnp.broadcast_to(An[:, None, :, None],
                                      (C, G, C, G)).reshape(GC, GC)
            An_ref[l] = jnp.where(mask, An_rep, 0.0).astype(jnp.bfloat16)

    x2 = x2_ref[...]
    xcm = jnp.concatenate(
        [x2[:, c * F:(c + 1) * F] for c in range(C)],
        axis=0).astype(jnp.bfloat16)
    h = jnp.maximum(
        jnp.dot(xcm, W_in_ref[...], preferred_element_type=jnp.float32)
        + b_in_ref[...], 0.0).astype(jnp.bfloat16)
    hw_ref[...] = h.reshape(GC, Tg * H)

    for l in range(L):
        An_bd = An_ref[l]
        Wl = W_ref[l].astype(jnp.bfloat16)
        bl = b_ref[l:l + 1, :]

        def layer_slice(s, _, An_bd=An_bd, Wl=Wl, bl=bl):
            sl = pl.ds(s * H, H)
            m = jnp.dot(An_bd, hw_ref[:, sl],
                        preferred_element_type=jnp.float32)
            acc = jnp.dot(m.astype(jnp.bfloat16), Wl,
                          preferred_element_type=jnp.float32)
            hw_ref[:, sl] = jnp.maximum(acc + bl, 0.0).astype(jnp.bfloat16)
            return 0

        jax.lax.fori_loop(0, Tg, layer_slice, 0)

    pr = jax.lax.broadcasted_iota(jnp.int32, (G, GC), 0)
    pc = jax.lax.broadcasted_iota(jnp.int32, (G, GC), 1)
    P = jnp.where(pr == pc % G, 1.0 / C, 0.0).astype(jnp.bfloat16)
    pooled = jnp.dot(P, hw_ref[...],
                     preferred_element_type=jnp.float32)
    feat = pooled.reshape(T, H).astype(jnp.bfloat16)
    out_ref[...] = (
        jnp.dot(feat, W_out_ref[...], preferred_element_type=jnp.float32)
        + b_out_ref[...])


def kernel(x, W_in, b_in, A, W, b, W_out, b_out):
    B, C, F = x.shape
    H = W_in.shape[1]
    K = W_out.shape[1]

    T = 1024
    G = _G
    Tg = T // G
    assert B % T == 0 and T % G == 0
    x2 = x.reshape(B, C * F)
    bf = jnp.bfloat16

    return pl.pallas_call(
        _body,
        grid=(B // T,),
        in_specs=[
            pl.BlockSpec((T, C * F), lambda i: (i, 0)),
            pl.BlockSpec(A.shape, lambda i: (0, 0, 0)),
            pl.BlockSpec(W_in.shape, lambda i: (0, 0)),
            pl.BlockSpec((1, H), lambda i: (0, 0)),
            pl.BlockSpec(W.shape, lambda i: (0, 0, 0)),
            pl.BlockSpec(b.shape, lambda i: (0, 0)),
            pl.BlockSpec(W_out.shape, lambda i: (0, 0)),
            pl.BlockSpec((1, K), lambda i: (0, 0)),
        ],
        out_specs=pl.BlockSpec((T, K), lambda i: (i, 0)),
        out_shape=jax.ShapeDtypeStruct((B, K), jnp.float32),
        scratch_shapes=[
            pltpu.VMEM((G * C, Tg * H), bf),
            pltpu.VMEM((3, G * C, G * C), bf),
        ],
        compiler_params=pltpu.CompilerParams(
            dimension_semantics=("arbitrary",)),
    )(x2, A, W_in.astype(bf), b_in.reshape(1, H), W.astype(bf), b,
      W_out.astype(bf), b_out.reshape(1, K))


# speedup vs baseline: 6.8739x; 6.8739x over previous
import jax
import jax.numpy as jnp
from jax.experimental import pallas as pl
from jax.experimental.pallas import tpu as pltpu

_G = 4


def _body(x2_ref, A_ref, W_in_ref, b_in_ref, W_ref, b_ref, W_out_ref,
          b_out_ref, out_ref, An_ref):
    T = x2_ref.shape[0]
    H = W_in_ref.shape[1]
    L, C, _ = A_ref.shape
    F = x2_ref.shape[1] // C
    G = _G
    Tg = T // G
    GC = G * C

    @pl.when(pl.program_id(0) == 0)
    def _build_adjacency():
        ri = jax.lax.broadcasted_iota(jnp.int32, (GC, GC), 0)
        ci = jax.lax.broadcasted_iota(jnp.int32, (GC, GC), 1)
        mask = (ri % G) == (ci % G)
        for l in range(L):
            a = A_ref[l].astype(jnp.float32)
            a = a - jnp.max(a, axis=-1, keepdims=True)
            e = jnp.exp(a)
            An = e / jnp.sum(e, axis=-1, keepdims=True)
            An_rep = jnp.broadcast_to(An[:, None, :, None],
                                      (C, G, C, G)).reshape(GC, GC)
            An_ref[l] = jnp.where(mask, An_rep, 0.0).astype(jnp.bfloat16)

    x2 = x2_ref[...]
    xcm = jnp.concatenate(
        [x2[:, c * F:(c + 1) * F] for c in range(C)],
        axis=0).astype(jnp.bfloat16)
    h = jnp.maximum(
        jnp.dot(xcm, W_in_ref[...], preferred_element_type=jnp.float32)
        + b_in_ref[...], 0.0).astype(jnp.bfloat16)

    for l in range(L):
        m = jnp.dot(An_ref[l], h.reshape(GC, Tg * H),
                    preferred_element_type=jnp.float32)
        h = jnp.maximum(
            jnp.dot(m.astype(jnp.bfloat16).reshape(GC * Tg, H), W_ref[l],
                    preferred_element_type=jnp.float32) + b_ref[l],
            0.0).astype(jnp.bfloat16)

    z = jnp.dot(h, W_out_ref[...],
                preferred_element_type=jnp.float32)
    out_ref[...] = (jnp.mean(z.reshape(C, T, out_ref.shape[1]), axis=0)
                    + b_out_ref[...])


def kernel(x, W_in, b_in, A, W, b, W_out, b_out):
    B, C, F = x.shape
    H = W_in.shape[1]
    K = W_out.shape[1]

    T = 1024
    assert B % T == 0 and T % _G == 0
    x2 = x.reshape(B, C * F)
    bf = jnp.bfloat16

    return pl.pallas_call(
        _body,
        grid=(B // T,),
        in_specs=[
            pl.BlockSpec((T, C * F), lambda i: (i, 0)),
            pl.BlockSpec(A.shape, lambda i: (0, 0, 0)),
            pl.BlockSpec(W_in.shape, lambda i: (0, 0)),
            pl.BlockSpec((1, H), lambda i: (0, 0)),
            pl.BlockSpec(W.shape, lambda i: (0, 0, 0)),
            pl.BlockSpec(b.shape, lambda i: (0, 0)),
            pl.BlockSpec(W_out.shape, lambda i: (0, 0)),
            pl.BlockSpec((1, K), lambda i: (0, 0)),
        ],
        out_specs=pl.BlockSpec((T, K), lambda i: (i, 0)),
        out_shape=jax.ShapeDtypeStruct((B, K), jnp.float32),
        scratch_shapes=[
            pltpu.VMEM((3, _G * C, _G * C), bf),
        ],
        compiler_params=pltpu.CompilerParams(
            dimension_semantics=("arbitrary",)),
    )(x2, A, W_in.astype(bf), b_in.reshape(1, H), W.astype(bf), b,
      W_out.astype(bf), b_out.reshape(1, K))


# speedup vs baseline: 7.3311x; 1.0665x over previous
import jax
import jax.numpy as jnp
from jax.experimental import pallas as pl
from jax.experimental.pallas import tpu as pltpu

_G = 4


def _body(xg_ref, A_ref, W_in_ref, b_in_ref, W_ref, b_ref, W_out_ref,
          b_out_ref, out_ref, An_ref):
    GC, Tg, F = xg_ref.shape
    H = W_in_ref.shape[1]
    L = A_ref.shape[0]
    C = A_ref.shape[1]
    G = GC // C
    K = out_ref.shape[1]
    T = G * Tg

    @pl.when(pl.program_id(0) == 0)
    def _build_adjacency():
        row_g = jax.lax.broadcasted_iota(jnp.int32, (GC, GC), 0) // C
        col_g = jax.lax.broadcasted_iota(jnp.int32, (GC, GC), 1) // C
        diag = row_g == col_g
        for l in range(L):
            a = A_ref[l].astype(jnp.float32)
            a = a - jnp.max(a, axis=-1, keepdims=True)
            e = jnp.exp(a)
            An = e / jnp.sum(e, axis=-1, keepdims=True)
            An_ref[l] = jnp.where(
                diag, jnp.tile(An, (G, G)), 0.0).astype(jnp.bfloat16)

    x = xg_ref[...].reshape(GC * Tg, F).astype(jnp.bfloat16)
    h = jnp.maximum(
        jnp.dot(x, W_in_ref[...], preferred_element_type=jnp.float32)
        + b_in_ref[...], 0.0).astype(jnp.bfloat16)

    for l in range(L):
        m = jnp.dot(An_ref[l], h.reshape(GC, Tg * H),
                    preferred_element_type=jnp.float32)
        h = jnp.maximum(
            jnp.dot(m.astype(jnp.bfloat16).reshape(GC * Tg, H), W_ref[l],
                    preferred_element_type=jnp.float32) + b_ref[l],
            0.0).astype(jnp.bfloat16)

    z = jnp.dot(h, W_out_ref[...],
                preferred_element_type=jnp.float32)
    feat = jnp.mean(z.reshape(G, C, Tg, K), axis=1)
    out_ref[...] = feat.reshape(T, K) + b_out_ref[...]


def kernel(x, W_in, b_in, A, W, b, W_out, b_out):
    B, C, F = x.shape
    H = W_in.shape[1]
    K = W_out.shape[1]

    T = 1024
    G = _G
    Tg = T // G
    assert B % T == 0 and T % G == 0
    n_tiles = B // T
    bf = jnp.bfloat16

    xg = jnp.transpose(x.reshape(n_tiles, G, Tg, C, F), (0, 1, 3, 2, 4))
    xg = xg.reshape(n_tiles * G * C, Tg, F)

    return pl.pallas_call(
        _body,
        grid=(n_tiles,),
        in_specs=[
            pl.BlockSpec((G * C, Tg, F), lambda i: (i, 0, 0)),
            pl.BlockSpec(A.shape, lambda i: (0, 0, 0)),
            pl.BlockSpec(W_in.shape, lambda i: (0, 0)),
            pl.BlockSpec((1, H), lambda i: (0, 0)),
            pl.BlockSpec(W.shape, lambda i: (0, 0, 0)),
            pl.BlockSpec(b.shape, lambda i: (0, 0)),
            pl.BlockSpec(W_out.shape, lambda i: (0, 0)),
            pl.BlockSpec((1, K), lambda i: (0, 0)),
        ],
        out_specs=pl.BlockSpec((T, K), lambda i: (i, 0)),
        out_shape=jax.ShapeDtypeStruct((B, K), jnp.float32),
        scratch_shapes=[
            pltpu.VMEM((3, G * C, G * C), bf),
        ],
        compiler_params=pltpu.CompilerParams(
            dimension_semantics=("arbitrary",)),
    )(xg, A, W_in.astype(bf), b_in.reshape(1, H), W.astype(bf), b,
      W_out.astype(bf), b_out.reshape(1, K))


# speedup vs baseline: 7.4753x; 1.0197x over previous
import jax
import jax.numpy as jnp
from jax.experimental import pallas as pl

_G = 4


def _body(xg_ref, A_ref, W_in_ref, b_in_ref, W_ref, b_ref, W_out_ref,
          b_out_ref, out_ref):
    GC, Tg, F = xg_ref.shape
    H = W_in_ref.shape[1]
    L = A_ref.shape[0]
    C = A_ref.shape[1]
    G = GC // C

    x = xg_ref[...].reshape(GC * Tg, F).astype(jnp.bfloat16)
    h = jnp.maximum(
        jnp.dot(x, W_in_ref[...].astype(jnp.bfloat16),
                preferred_element_type=jnp.float32)
        + b_in_ref[...], 0.0)

    row_g = jax.lax.broadcasted_iota(jnp.int32, (GC, GC), 0) // C
    col_g = jax.lax.broadcasted_iota(jnp.int32, (GC, GC), 1) // C
    diag = row_g == col_g

    for l in range(L):
        a = A_ref[l]
        a = a - jnp.max(a, axis=-1, keepdims=True)
        e = jnp.exp(a)
        An = e / jnp.sum(e, axis=-1, keepdims=True)
        An_bd = jnp.where(diag, jnp.tile(An, (G, G)), 0.0)
        m = jnp.dot(An_bd.astype(jnp.bfloat16),
                    h.astype(jnp.bfloat16).reshape(GC, Tg * H),
                    preferred_element_type=jnp.float32)
        h = jnp.maximum(
            jnp.dot(m.reshape(GC * Tg, H).astype(jnp.bfloat16),
                    W_ref[l].astype(jnp.bfloat16),
                    preferred_element_type=jnp.float32) + b_ref[l], 0.0)

    feat = jnp.mean(h.reshape(G, C, Tg, H), axis=1)
    out_ref[...] = (
        jnp.dot(feat.reshape(G * Tg, H).astype(jnp.bfloat16),
                W_out_ref[...].astype(jnp.bfloat16),
                preferred_element_type=jnp.float32) + b_out_ref[...])


def kernel(x, W_in, b_in, A, W, b, W_out, b_out):
    B, C, F = x.shape
    H = W_in.shape[1]
    K = W_out.shape[1]

    T = 1024
    G = _G
    Tg = T // G
    assert B % T == 0 and T % G == 0
    n_tiles = B // T

    xg = jnp.transpose(x.reshape(n_tiles, G, Tg, C, F), (0, 1, 3, 2, 4))
    xg = xg.reshape(n_tiles * G * C, Tg, F)

    return pl.pallas_call(
        _body,
        grid=(n_tiles,),
        in_specs=[
            pl.BlockSpec((G * C, Tg, F), lambda i: (i, 0, 0)),
            pl.BlockSpec(A.shape, lambda i: (0, 0, 0)),
            pl.BlockSpec(W_in.shape, lambda i: (0, 0)),
            pl.BlockSpec((1, H), lambda i: (0, 0)),
            pl.BlockSpec(W.shape, lambda i: (0, 0, 0)),
            pl.BlockSpec(b.shape, lambda i: (0, 0)),
            pl.BlockSpec(W_out.shape, lambda i: (0, 0)),
            pl.BlockSpec((1, K), lambda i: (0, 0)),
        ],
        out_specs=pl.BlockSpec((T, K), lambda i: (i, 0)),
        out_shape=jax.ShapeDtypeStruct((B, K), jnp.float32),
    )(xg, A, W_in, b_in.reshape(1, H), W, b, W_out, b_out.reshape(1, K))
